# Initial kernel scaffold; baseline (speedup 1.0000x reference)
import jax
import jax.numpy as jnp
from jax.experimental import pallas as pl


def kernel(node_features, edge_index, edge_src, edge_dst, W_self1, W_neigh1, b1, W_self2, W_neigh2, b2, W_pred, b_pred):
    raise NotImplementedError("write your pallas kernel here")


# speedup vs baseline: 5.6179x; 5.6179x over previous
import dataclasses
import functools

import jax
import jax.numpy as jnp
from jax import lax
from jax.experimental import pallas as pl
from jax.experimental.pallas import tpu as pltpu
from jax.experimental.pallas import tpu_sc as plsc

N_NODES = 10000
N_EDGES = 320000
F = 128
H = 256

NC = 2
NS = 16
NW = NC * NS
L = 16

CHUNK = 128
K_CHUNKS = 80
EDGES_PER_TILE = CHUNK * K_CHUNKS
E_PAD = EDGES_PER_TILE * NW
NPAD = 10112
ROWS_PER_TILE = NPAD // NS
TRASH = N_NODES

@functools.cache
def _vmesh():
    return plsc.VectorSubcoreMesh(core_axis_name="c", subcore_axis_name="s",
                                  num_cores=NC, num_subcores=NS)


def _f32(*shape):
    return jax.ShapeDtypeStruct(shape, jnp.float32)


@functools.cache
def _make_segsum():
    scratch = [
        pltpu.VMEM((2, 2, CHUNK), jnp.int32),
        pltpu.VMEM((2, CHUNK, F), jnp.float32),
        pltpu.VMEM_SHARED((NPAD, F), jnp.float32),
        pltpu.SemaphoreType.DMA,
        pltpu.SemaphoreType.DMA,
    ]

    def body(table_hbm, src_hbm, dst_hbm, zf_hbm, agg_hbm,
             ibuf, rows, acc_sh, sem0, sem1):
        sems = (sem0, sem1)
        c = lax.axis_index("c")
        s = lax.axis_index("s")
        w = c * NS + s
        ebase = w * EDGES_PER_TILE
        rbase = s * ROWS_PER_TILE

        pltpu.sync_copy(zf_hbm.at[pl.ds(rbase, ROWS_PER_TILE)],
                        acc_sh.at[pl.ds(rbase, ROWS_PER_TILE)])
        plsc.subcore_barrier()

        def prefetch(chunk, b):
            off = ebase + chunk * CHUNK
            pltpu.sync_copy(src_hbm.at[pl.ds(off, CHUNK)], ibuf.at[b, 0])
            pltpu.sync_copy(dst_hbm.at[pl.ds(off, CHUNK)], ibuf.at[b, 1])
            pltpu.async_copy(table_hbm.at[ibuf.at[b, 0]], rows.at[b], sems[b])

        def drain(b):
            pltpu.make_async_copy(
                table_hbm.at[ibuf.at[b, 0]], rows.at[b], sems[b]).wait()
            pltpu.sync_copy(rows.at[b], acc_sh.at[ibuf.at[b, 1]], add=True)

        prefetch(0, 0)
        prefetch(1, 1)

        @pl.loop(0, K_CHUNKS - 2, step=2)
        def _(k):
            for b in range(2):
                drain(b)
                prefetch(k + 2 + b, b)

        drain(0)
        drain(1)
        plsc.subcore_barrier()

        pltpu.sync_copy(acc_sh.at[pl.ds(rbase, ROWS_PER_TILE)],
                        agg_hbm.at[c, pl.ds(rbase, ROWS_PER_TILE)])

    return pl.kernel(body, out_type=_f32(NC, NPAD, F), mesh=_vmesh(),
                     scratch_types=scratch, name="segsum")


_KB = 40


@functools.cache
def _make_deg():
    def body(dst_hbm, zf_hbm, od_hbm, deg_hbm, dbuf, ones_v, deg_sh,
             sem0, sem1):
        sems = (sem0, sem1)
        c = lax.axis_index("c")
        s = lax.axis_index("s")
        w = c * NS + s
        ebase = w * EDGES_PER_TILE
        rbase = s * ROWS_PER_TILE

        pltpu.sync_copy(zf_hbm.at[pl.ds(rbase, ROWS_PER_TILE)],
                        deg_sh.at[pl.ds(rbase, ROWS_PER_TILE)])
        pltpu.sync_copy(od_hbm, ones_v)
        plsc.subcore_barrier()

        def prefetch(chunk, b):
            off = ebase + chunk * CHUNK
            pltpu.async_copy(dst_hbm.at[pl.ds(off, CHUNK)], dbuf.at[b],
                             sems[b])

        def drain(b):
            pltpu.make_async_copy(dst_hbm.at[pl.ds(0, CHUNK)], dbuf.at[b],
                                  sems[b]).wait()
            pltpu.sync_copy(ones_v, deg_sh.at[dbuf.at[b]], add=True)

        prefetch(0, 0)
        prefetch(1, 1)

        @pl.loop(0, K_CHUNKS - 2, step=2)
        def _(k):
            for b in range(2):
                drain(b)
                prefetch(k + 2 + b, b)

        drain(0)
        drain(1)
        plsc.subcore_barrier()

        pltpu.sync_copy(deg_sh.at[pl.ds(rbase, ROWS_PER_TILE)],
                        deg_hbm.at[c, pl.ds(rbase, ROWS_PER_TILE)])

    return pl.kernel(
        body, out_type=_f32(NC, NPAD, F), mesh=_vmesh(),
        scratch_types=[
            pltpu.VMEM((2, CHUNK), jnp.int32),
            pltpu.VMEM((CHUNK, F), jnp.float32),
            pltpu.VMEM_SHARED((NPAD, F), jnp.float32),
            pltpu.SemaphoreType.DMA,
            pltpu.SemaphoreType.DMA,
        ],
        name="deg_histogram")


def _predict(sv_hbm_arr, tv_hbm_arr, esrc, edst):
    epw = N_EDGES // NW

    def body(s_hbm, t_hbm, es_hbm, ed_hbm, out_hbm, s_v, t_v, es_v, ed_v,
             out_v):
        c = lax.axis_index("c")
        s = lax.axis_index("s")
        w = c * NS + s
        base = w * epw
        pltpu.sync_copy(s_hbm, s_v)
        pltpu.sync_copy(t_hbm, t_v)
        pltpu.sync_copy(es_hbm.at[pl.ds(base, epw)], es_v)
        pltpu.sync_copy(ed_hbm.at[pl.ds(base, epw)], ed_v)

        @pl.loop(0, epw, step=L)
        def _(i):
            si = es_v[pl.ds(i, L)]
            di = ed_v[pl.ds(i, L)]
            sv = plsc.load_gather(s_v, [si])
            tv = plsc.load_gather(t_v, [di])
            out_v[pl.ds(i, L)] = sv + tv

        pltpu.sync_copy(out_v, out_hbm.at[pl.ds(base, epw)])

    cp = pltpu.CompilerParams()
    if "needs_layout_passes" in pltpu.CompilerParams.__dataclass_fields__:
        cp = dataclasses.replace(cp, needs_layout_passes=False)
    return pl.kernel(
        body, out_type=_f32(N_EDGES), mesh=_vmesh(),
        scratch_types=[
            pltpu.VMEM((N_NODES,), jnp.float32),
            pltpu.VMEM((N_NODES,), jnp.float32),
            pltpu.VMEM((epw,), jnp.int32),
            pltpu.VMEM((epw,), jnp.int32),
            pltpu.VMEM((epw,), jnp.float32),
        ],
        compiler_params=cp,
        name="edge_predict")(sv_hbm_arr, tv_hbm_arr, esrc, edst)


_R = 400
_GRID = N_NODES // _R


def _recip_deg(dga, dgb):
    deg = dga[0][:, 0:1] + dgb[0][:, 0:1]
    return 1.0 / jnp.maximum(deg, 1.0)


def _layer_body(x, a1a, a1b, dga, dgb, ws1, wn1, b1, ws2, wn2, b2, y, z):
    mean = (a1a[0] + a1b[0]) * _recip_deg(dga, dgb)
    h = jnp.dot(x[...], ws1[...], preferred_element_type=jnp.float32)
    h += jnp.dot(mean, wn1[...], preferred_element_type=jnp.float32)
    h = jnp.maximum(h + b1[0], 0.0)
    y[...] = jnp.dot(h, wn2[...], preferred_element_type=jnp.float32)
    z[...] = (jnp.dot(h, ws2[...], preferred_element_type=jnp.float32)
              + b2[0])


def _layers(x, agg1, deg, ws1, wn1, b1, ws2, wn2, b2):
    part = lambda core: pl.BlockSpec((1, _R, F), lambda i, c=core: (c, i, 0))
    dpart = lambda core: pl.BlockSpec((1, _R, F), lambda i, c=core: (c, i, 0))
    full = lambda *blk: pl.BlockSpec(blk, lambda i: (0,) * len(blk))
    return pl.pallas_call(
        _layer_body,
        grid=(_GRID,),
        in_specs=[
            pl.BlockSpec((_R, F), lambda i: (i, 0)),
            part(0), part(1), dpart(0), dpart(1),
            full(F, H), full(F, H), full(1, H),
            full(H, F), full(H, F), full(1, F),
        ],
        out_specs=[pl.BlockSpec((_R, F), lambda i: (i, 0))] * 2,
        out_shape=[_f32(N_NODES, F)] * 2,
    )(x, agg1, agg1, deg, deg, ws1, wn1, b1, ws2, wn2, b2)


def _score_body(z, a2a, a2b, dga, dgb, wp2, bp, st):
    h2 = z[...] + (a2a[0] + a2b[0]) * _recip_deg(dga, dgb)
    out = jnp.dot(h2, wp2[...].T, preferred_element_type=jnp.float32)
    is_s = (lax.broadcasted_iota(jnp.int32, (1, 2), 1) == 0)
    st[...] = out + bp[0, 0] * is_s.astype(jnp.float32)


def _scores(z, agg2, deg, wp2, bp):
    part = lambda core: pl.BlockSpec((1, _R, F), lambda i, c=core: (c, i, 0))
    dpart = lambda core: pl.BlockSpec((1, _R, F), lambda i, c=core: (c, i, 0))
    return pl.pallas_call(
        _score_body,
        grid=(_GRID,),
        in_specs=[
            pl.BlockSpec((_R, F), lambda i: (i, 0)),
            part(0), part(1), dpart(0), dpart(1),
            pl.BlockSpec((2, F), lambda i: (0, 0)),
            pl.BlockSpec((1, 1), lambda i: (0, 0)),
        ],
        out_specs=pl.BlockSpec((_R, 2), lambda i: (i, 0)),
        out_shape=_f32(N_NODES, 2),
    )(z, agg2, agg2, deg, deg, wp2, bp)


def kernel(node_features, edge_index, edge_src, edge_dst,
           W_self1, W_neigh1, b1, W_self2, W_neigh2, b2, W_pred, b_pred):
    pad = E_PAD - N_EDGES
    src = jnp.concatenate(
        [edge_index[0].astype(jnp.int32), jnp.zeros((pad,), jnp.int32)])
    dst = jnp.concatenate(
        [edge_index[1].astype(jnp.int32), jnp.full((pad,), TRASH, jnp.int32)])
    esrc = edge_src.astype(jnp.int32)
    edst = edge_dst.astype(jnp.int32)

    zf = jnp.zeros((NPAD, F), jnp.float32)
    od = jnp.ones((CHUNK, F), jnp.float32)
    agg1 = _make_segsum()(node_features, src, dst, zf)
    deg = _make_deg()(dst, zf, od)
    y, z = _layers(node_features, agg1, deg,
                   W_self1, W_neigh1, b1.reshape(1, H),
                   W_self2, W_neigh2, b2.reshape(1, F))
    agg2 = _make_segsum()(y, src, dst, zf)
    st = _scores(z, agg2, deg, W_pred.reshape(2, F), b_pred.reshape(1, 1))
    return _predict(st[:, 0], st[:, 1], esrc, edst)


# speedup vs baseline: 12.3438x; 2.1972x over previous
import dataclasses
import functools

import jax
import jax.numpy as jnp
from jax import lax
from jax.experimental import pallas as pl
from jax.experimental.pallas import tpu as pltpu
from jax.experimental.pallas import tpu_sc as plsc

N_NODES = 10000
N_EDGES = 320000
F = 128
H = 256

NC = 2
NS = 16
NW = NC * NS
L = 16

CHUNK = 128
K_CHUNKS = 80
EDGES_PER_TILE = CHUNK * K_CHUNKS
E_PAD = EDGES_PER_TILE * NW
NPAD = 10112
ROWS_PER_TILE = NPAD // NS
TRASH = N_NODES

@functools.cache
def _vmesh():
    return plsc.VectorSubcoreMesh(core_axis_name="c", subcore_axis_name="s",
                                  num_cores=NC, num_subcores=NS)


def _f32(*shape):
    return jax.ShapeDtypeStruct(shape, jnp.float32)


@functools.cache
def _make_segsum():
    scratch = [
        pltpu.VMEM((2, 2, CHUNK), jnp.int32),
        pltpu.VMEM((2, CHUNK, F), jnp.float32),
        pltpu.VMEM_SHARED((NPAD, F), jnp.float32),
        pltpu.SemaphoreType.DMA,
        pltpu.SemaphoreType.DMA,
    ]

    def body(table_hbm, src_hbm, dst_hbm, zf_hbm, agg_hbm,
             ibuf, rows, acc_sh, sem0, sem1):
        sems = (sem0, sem1)
        c = lax.axis_index("c")
        s = lax.axis_index("s")
        w = c * NS + s
        ebase = w * EDGES_PER_TILE
        rbase = s * ROWS_PER_TILE

        pltpu.sync_copy(zf_hbm.at[pl.ds(rbase, ROWS_PER_TILE)],
                        acc_sh.at[pl.ds(rbase, ROWS_PER_TILE)])
        plsc.subcore_barrier()

        def prefetch(chunk, b):
            off = ebase + chunk * CHUNK
            pltpu.sync_copy(src_hbm.at[pl.ds(off, CHUNK)], ibuf.at[b, 0])
            pltpu.sync_copy(dst_hbm.at[pl.ds(off, CHUNK)], ibuf.at[b, 1])
            pltpu.async_copy(table_hbm.at[ibuf.at[b, 0]], rows.at[b], sems[b])

        def drain(b):
            pltpu.make_async_copy(
                table_hbm.at[ibuf.at[b, 0]], rows.at[b], sems[b]).wait()
            pltpu.sync_copy(rows.at[b], acc_sh.at[ibuf.at[b, 1]], add=True)

        prefetch(0, 0)
        prefetch(1, 1)

        @pl.loop(0, K_CHUNKS - 2, step=2)
        def _(k):
            for b in range(2):
                drain(b)
                prefetch(k + 2 + b, b)

        drain(0)
        drain(1)
        plsc.subcore_barrier()

        pltpu.sync_copy(acc_sh.at[pl.ds(rbase, ROWS_PER_TILE)],
                        agg_hbm.at[c, pl.ds(rbase, ROWS_PER_TILE)])

    return pl.kernel(body, out_type=_f32(NC, NPAD, F), mesh=_vmesh(),
                     scratch_types=scratch, name="segsum")


_KB = 40


@functools.cache
def _make_deg():
    def body(dst_hbm, zf_hbm, od_hbm, deg_hbm, dbuf, ones_v, deg_sh,
             sem0, sem1):
        sems = (sem0, sem1)
        c = lax.axis_index("c")
        s = lax.axis_index("s")
        w = c * NS + s
        ebase = w * EDGES_PER_TILE
        rbase = s * ROWS_PER_TILE

        pltpu.sync_copy(zf_hbm.at[pl.ds(rbase, ROWS_PER_TILE)],
                        deg_sh.at[pl.ds(rbase, ROWS_PER_TILE)])
        pltpu.sync_copy(od_hbm, ones_v)
        plsc.subcore_barrier()

        def prefetch(chunk, b):
            off = ebase + chunk * CHUNK
            pltpu.async_copy(dst_hbm.at[pl.ds(off, CHUNK)], dbuf.at[b],
                             sems[b])

        def drain(b):
            pltpu.make_async_copy(dst_hbm.at[pl.ds(0, CHUNK)], dbuf.at[b],
                                  sems[b]).wait()
            pltpu.sync_copy(ones_v, deg_sh.at[dbuf.at[b]], add=True)

        prefetch(0, 0)
        prefetch(1, 1)

        @pl.loop(0, K_CHUNKS - 2, step=2)
        def _(k):
            for b in range(2):
                drain(b)
                prefetch(k + 2 + b, b)

        drain(0)
        drain(1)
        plsc.subcore_barrier()

        pltpu.sync_copy(deg_sh.at[pl.ds(rbase, ROWS_PER_TILE)],
                        deg_hbm.at[c, pl.ds(rbase, ROWS_PER_TILE)])

    return pl.kernel(
        body, out_type=_f32(NC, NPAD, F), mesh=_vmesh(),
        scratch_types=[
            pltpu.VMEM((2, CHUNK), jnp.int32),
            pltpu.VMEM((CHUNK, F), jnp.float32),
            pltpu.VMEM_SHARED((NPAD, F), jnp.float32),
            pltpu.SemaphoreType.DMA,
            pltpu.SemaphoreType.DMA,
        ],
        name="deg_histogram")


def _predict(sv_hbm_arr, tv_hbm_arr, esrc, edst):
    epw = N_EDGES // NW

    def body(s_hbm, t_hbm, es_hbm, ed_hbm, out_hbm, s_v, t_v, es_v, ed_v,
             out_v):
        c = lax.axis_index("c")
        s = lax.axis_index("s")
        w = c * NS + s
        base = w * epw
        pltpu.sync_copy(s_hbm, s_v)
        pltpu.sync_copy(t_hbm, t_v)
        pltpu.sync_copy(es_hbm.at[pl.ds(base, epw)], es_v)
        pltpu.sync_copy(ed_hbm.at[pl.ds(base, epw)], ed_v)

        @pl.loop(0, epw, step=L)
        def _(i):
            si = es_v[pl.ds(i, L)]
            di = ed_v[pl.ds(i, L)]
            sv = plsc.load_gather(s_v, [si])
            tv = plsc.load_gather(t_v, [di])
            out_v[pl.ds(i, L)] = sv + tv

        pltpu.sync_copy(out_v, out_hbm.at[pl.ds(base, epw)])

    cp = pltpu.CompilerParams()
    if "needs_layout_passes" in pltpu.CompilerParams.__dataclass_fields__:
        cp = dataclasses.replace(cp, needs_layout_passes=False)
    return pl.kernel(
        body, out_type=_f32(N_EDGES), mesh=_vmesh(),
        scratch_types=[
            pltpu.VMEM((N_NODES,), jnp.float32),
            pltpu.VMEM((N_NODES,), jnp.float32),
            pltpu.VMEM((epw,), jnp.int32),
            pltpu.VMEM((epw,), jnp.int32),
            pltpu.VMEM((epw,), jnp.float32),
        ],
        compiler_params=cp,
        name="edge_predict")(sv_hbm_arr, tv_hbm_arr, esrc, edst)


_R = 400
_GRID = N_NODES // _R


def _recip_deg(dga, dgb):
    deg = dga[0][:, 0:1] + dgb[0][:, 0:1]
    return 1.0 / jnp.maximum(deg, 1.0)


def _layer_body(x, a1a, a1b, dga, dgb, ws1, wn1, b1, ws2, wn2, b2, y, z):
    mean = (a1a[0] + a1b[0]) * _recip_deg(dga, dgb)
    h = jnp.dot(x[...], ws1[...], preferred_element_type=jnp.float32)
    h += jnp.dot(mean, wn1[...], preferred_element_type=jnp.float32)
    h = jnp.maximum(h + b1[0], 0.0)
    y[...] = jnp.dot(h, wn2[...], preferred_element_type=jnp.float32)
    z[...] = (jnp.dot(h, ws2[...], preferred_element_type=jnp.float32)
              + b2[0])


def _layers(x, agg1, deg, ws1, wn1, b1, ws2, wn2, b2):
    part = lambda core: pl.BlockSpec((1, _R, F), lambda i, c=core: (c, i, 0))
    dpart = lambda core: pl.BlockSpec((1, _R, F), lambda i, c=core: (c, i, 0))
    full = lambda *blk: pl.BlockSpec(blk, lambda i: (0,) * len(blk))
    return pl.pallas_call(
        _layer_body,
        grid=(_GRID,),
        in_specs=[
            pl.BlockSpec((_R, F), lambda i: (i, 0)),
            part(0), part(1), dpart(0), dpart(1),
            full(F, H), full(F, H), full(1, H),
            full(H, F), full(H, F), full(1, F),
        ],
        out_specs=[pl.BlockSpec((_R, F), lambda i: (i, 0))] * 2,
        out_shape=[_f32(N_NODES, F)] * 2,
    )(x, agg1, agg1, deg, deg, ws1, wn1, b1, ws2, wn2, b2)


def _score_body(z, a2a, a2b, dga, dgb, wp2, bp, st):
    h2 = z[...] + (a2a[0] + a2b[0]) * _recip_deg(dga, dgb)
    out = jnp.dot(h2, wp2[...].T, preferred_element_type=jnp.float32)
    is_s = (lax.broadcasted_iota(jnp.int32, (1, 2), 1) == 0)
    st[...] = out + bp[0, 0] * is_s.astype(jnp.float32)


def _scores(z, agg2, deg, wp2, bp):
    part = lambda core: pl.BlockSpec((1, _R, F), lambda i, c=core: (c, i, 0))
    dpart = lambda core: pl.BlockSpec((1, _R, F), lambda i, c=core: (c, i, 0))
    return pl.pallas_call(
        _score_body,
        grid=(_GRID,),
        in_specs=[
            pl.BlockSpec((_R, F), lambda i: (i, 0)),
            part(0), part(1), dpart(0), dpart(1),
            pl.BlockSpec((2, F), lambda i: (0, 0)),
            pl.BlockSpec((1, 1), lambda i: (0, 0)),
        ],
        out_specs=pl.BlockSpec((_R, 2), lambda i: (i, 0)),
        out_shape=_f32(N_NODES, 2),
    )(z, agg2, agg2, deg, deg, wp2, bp)


def kernel(node_features, edge_index, edge_src, edge_dst,
           W_self1, W_neigh1, b1, W_self2, W_neigh2, b2, W_pred, b_pred):
    pad = E_PAD - N_EDGES
    pad_src = jnp.arange(pad, dtype=jnp.int32) % N_NODES
    pad_dst = TRASH + jnp.arange(pad, dtype=jnp.int32) % (NPAD - TRASH)
    src = jnp.concatenate([edge_index[0].astype(jnp.int32), pad_src])
    dst = jnp.concatenate([edge_index[1].astype(jnp.int32), pad_dst])
    esrc = edge_src.astype(jnp.int32)
    edst = edge_dst.astype(jnp.int32)

    zf = jnp.zeros((NPAD, F), jnp.float32)
    od = jnp.ones((CHUNK, F), jnp.float32)
    agg1 = _make_segsum()(node_features, src, dst, zf)
    deg = _make_deg()(dst, zf, od)
    y, z = _layers(node_features, agg1, deg,
                   W_self1, W_neigh1, b1.reshape(1, H),
                   W_self2, W_neigh2, b2.reshape(1, F))
    agg2 = _make_segsum()(y, src, dst, zf)
    st = _scores(z, agg2, deg, W_pred.reshape(2, F), b_pred.reshape(1, 1))
    return _predict(st[:, 0], st[:, 1], esrc, edst)


# speedup vs baseline: 14.8398x; 1.2022x over previous
import dataclasses
import functools

import jax
import jax.numpy as jnp
from jax import lax
from jax.experimental import pallas as pl
from jax.experimental.pallas import tpu as pltpu
from jax.experimental.pallas import tpu_sc as plsc

N_NODES = 10000
N_EDGES = 320000
F = 128
H = 256

NC = 2
NS = 16
NW = NC * NS
L = 16

CHUNK = 128
K_CHUNKS = 80
EDGES_PER_TILE = CHUNK * K_CHUNKS
E_PAD = EDGES_PER_TILE * NW
NPAD = 10112
ROWS_PER_TILE = NPAD // NS
TRASH = N_NODES

@functools.cache
def _vmesh():
    return plsc.VectorSubcoreMesh(core_axis_name="c", subcore_axis_name="s",
                                  num_cores=NC, num_subcores=NS)


def _f32(*shape):
    return jax.ShapeDtypeStruct(shape, jnp.float32)


@functools.cache
def _make_segsum(width, chunk):
    nk = EDGES_PER_TILE // chunk
    scratch = [
        pltpu.VMEM((4, 2, chunk), jnp.int32),
        pltpu.VMEM((2, chunk, width), jnp.float32),
        pltpu.VMEM_SHARED((NPAD, width), jnp.float32),
    ] + [pltpu.SemaphoreType.DMA] * 6

    def body(table_hbm, sd_hbm, zf_hbm, agg_hbm,
             ibuf, rows, acc_sh, g0, g1, i0, i1, i2, i3):
        gsems = (g0, g1)
        isems = (i0, i1, i2, i3)
        c = lax.axis_index("c")
        s = lax.axis_index("s")
        w = c * NS + s
        cg0 = w * nk
        rbase = s * ROWS_PER_TILE

        pltpu.sync_copy(zf_hbm.at[pl.ds(rbase, ROWS_PER_TILE)],
                        acc_sh.at[pl.ds(rbase, ROWS_PER_TILE)])
        plsc.subcore_barrier()

        def start_idx(k, q):
            pltpu.async_copy(sd_hbm.at[cg0 + k], ibuf.at[q], isems[q])

        def wait_idx(q):
            pltpu.make_async_copy(sd_hbm.at[0], ibuf.at[q], isems[q]).wait()

        def start_g(q, b):
            pltpu.async_copy(table_hbm.at[ibuf.at[q, 0]], rows.at[b],
                             gsems[b])

        def drain(q, b):
            pltpu.make_async_copy(table_hbm.at[ibuf.at[q, 0]], rows.at[b],
                                  gsems[b]).wait()
            pltpu.sync_copy(rows.at[b], acc_sh.at[ibuf.at[q, 1]], add=True)

        for q in range(4):
            start_idx(q, q)
        wait_idx(0)
        wait_idx(1)
        start_g(0, 0)
        start_g(1, 1)

        def step(k, j, last):
            drain(j, j % 2)
            if not last or j < 2:
                wait_idx((j + 2) % 4)
                start_g((j + 2) % 4, j % 2)
            if not last:
                start_idx(k + j + 4, j)

        @pl.loop(0, nk - 4, step=4)
        def _(k):
            for j in range(4):
                step(k, j, False)

        for j in range(4):
            step(nk - 4, j, True)
        plsc.subcore_barrier()

        pltpu.sync_copy(acc_sh.at[pl.ds(rbase, ROWS_PER_TILE)],
                        agg_hbm.at[c, pl.ds(rbase, ROWS_PER_TILE)])

    return pl.kernel(body, out_type=_f32(NC, NPAD, width), mesh=_vmesh(),
                     scratch_types=scratch, name=f"segsum{width}")


_KB = 40


@functools.cache
def _make_deg():
    def body(dst_hbm, zf_hbm, od_hbm, deg_hbm, dbuf, ones_v, deg_sh,
             sem0, sem1):
        sems = (sem0, sem1)
        c = lax.axis_index("c")
        s = lax.axis_index("s")
        w = c * NS + s
        ebase = w * EDGES_PER_TILE
        rbase = s * ROWS_PER_TILE

        pltpu.sync_copy(zf_hbm.at[pl.ds(rbase, ROWS_PER_TILE)],
                        deg_sh.at[pl.ds(rbase, ROWS_PER_TILE)])
        pltpu.sync_copy(od_hbm, ones_v)
        plsc.subcore_barrier()

        def prefetch(chunk, b):
            off = ebase + chunk * CHUNK
            pltpu.async_copy(dst_hbm.at[pl.ds(off, CHUNK)], dbuf.at[b],
                             sems[b])

        def drain(b):
            pltpu.make_async_copy(dst_hbm.at[pl.ds(0, CHUNK)], dbuf.at[b],
                                  sems[b]).wait()
            pltpu.sync_copy(ones_v, deg_sh.at[dbuf.at[b]], add=True)

        prefetch(0, 0)
        prefetch(1, 1)

        @pl.loop(0, K_CHUNKS - 2, step=2)
        def _(k):
            for b in range(2):
                drain(b)
                prefetch(k + 2 + b, b)

        drain(0)
        drain(1)
        plsc.subcore_barrier()

        pltpu.sync_copy(deg_sh.at[pl.ds(rbase, ROWS_PER_TILE)],
                        deg_hbm.at[c, pl.ds(rbase, ROWS_PER_TILE)])

    return pl.kernel(
        body, out_type=_f32(NC, NPAD, F), mesh=_vmesh(),
        scratch_types=[
            pltpu.VMEM((2, CHUNK), jnp.int32),
            pltpu.VMEM((CHUNK, F), jnp.float32),
            pltpu.VMEM_SHARED((NPAD, F), jnp.float32),
            pltpu.SemaphoreType.DMA,
            pltpu.SemaphoreType.DMA,
        ],
        name="deg_histogram")


def _predict(sv_hbm_arr, tv_hbm_arr, esrc, edst):
    epw = N_EDGES // NW

    def body(s_hbm, t_hbm, es_hbm, ed_hbm, out_hbm, s_v, t_v, es_v, ed_v,
             out_v):
        c = lax.axis_index("c")
        s = lax.axis_index("s")
        w = c * NS + s
        base = w * epw
        pltpu.sync_copy(s_hbm, s_v)
        pltpu.sync_copy(t_hbm, t_v)
        pltpu.sync_copy(es_hbm.at[pl.ds(base, epw)], es_v)
        pltpu.sync_copy(ed_hbm.at[pl.ds(base, epw)], ed_v)

        @pl.loop(0, epw, step=L)
        def _(i):
            si = es_v[pl.ds(i, L)]
            di = ed_v[pl.ds(i, L)]
            sv = plsc.load_gather(s_v, [si])
            tv = plsc.load_gather(t_v, [di])
            out_v[pl.ds(i, L)] = sv + tv

        pltpu.sync_copy(out_v, out_hbm.at[pl.ds(base, epw)])

    cp = pltpu.CompilerParams()
    if "needs_layout_passes" in pltpu.CompilerParams.__dataclass_fields__:
        cp = dataclasses.replace(cp, needs_layout_passes=False)
    return pl.kernel(
        body, out_type=_f32(N_EDGES), mesh=_vmesh(),
        scratch_types=[
            pltpu.VMEM((N_NODES,), jnp.float32),
            pltpu.VMEM((N_NODES,), jnp.float32),
            pltpu.VMEM((epw,), jnp.int32),
            pltpu.VMEM((epw,), jnp.int32),
            pltpu.VMEM((epw,), jnp.float32),
        ],
        compiler_params=cp,
        name="edge_predict")(sv_hbm_arr, tv_hbm_arr, esrc, edst)


_R = 400
_GRID = N_NODES // _R


def _recip_deg(dga, dgb):
    deg = dga[0][:, 0:1] + dgb[0][:, 0:1]
    return 1.0 / jnp.maximum(deg, 1.0)


def _layer_body(x, a1a, a1b, dga, dgb, ws1, wn1, b1, ws2, wn2, b2, y, z):
    mean = (a1a[0] + a1b[0]) * _recip_deg(dga, dgb)
    h = jnp.dot(x[...], ws1[...], preferred_element_type=jnp.float32)
    h += jnp.dot(mean, wn1[...], preferred_element_type=jnp.float32)
    h = jnp.maximum(h + b1[0], 0.0)
    y[...] = jnp.dot(h, wn2[...], preferred_element_type=jnp.float32)
    z[...] = (jnp.dot(h, ws2[...], preferred_element_type=jnp.float32)
              + b2[0])


def _layers(x, agg1, deg, ws1, wn1, b1, ws2, wn2, b2):
    part = lambda core: pl.BlockSpec((1, _R, F), lambda i, c=core: (c, i, 0))
    dpart = lambda core: pl.BlockSpec((1, _R, F), lambda i, c=core: (c, i, 0))
    full = lambda *blk: pl.BlockSpec(blk, lambda i: (0,) * len(blk))
    return pl.pallas_call(
        _layer_body,
        grid=(_GRID,),
        in_specs=[
            pl.BlockSpec((_R, F), lambda i: (i, 0)),
            part(0), part(1), dpart(0), dpart(1),
            full(F, H), full(F, H), full(1, H),
            full(H, F), full(H, F), full(1, F),
        ],
        out_specs=[pl.BlockSpec((_R, F), lambda i: (i, 0))] * 2,
        out_shape=[_f32(N_NODES, F)] * 2,
    )(x, agg1, agg1, deg, deg, ws1, wn1, b1, ws2, wn2, b2)


def _score_body(z, a2a, a2b, dga, dgb, wp2, bp, st):
    h2 = z[...] + (a2a[0] + a2b[0]) * _recip_deg(dga, dgb)
    out = jnp.dot(h2, wp2[...].T, preferred_element_type=jnp.float32)
    is_s = (lax.broadcasted_iota(jnp.int32, (1, 2), 1) == 0)
    st[...] = out + bp[0, 0] * is_s.astype(jnp.float32)


def _scores(z, agg2, deg, wp2, bp):
    part = lambda core: pl.BlockSpec((1, _R, F), lambda i, c=core: (c, i, 0))
    dpart = lambda core: pl.BlockSpec((1, _R, F), lambda i, c=core: (c, i, 0))
    return pl.pallas_call(
        _score_body,
        grid=(_GRID,),
        in_specs=[
            pl.BlockSpec((_R, F), lambda i: (i, 0)),
            part(0), part(1), dpart(0), dpart(1),
            pl.BlockSpec((2, F), lambda i: (0, 0)),
            pl.BlockSpec((1, 1), lambda i: (0, 0)),
        ],
        out_specs=pl.BlockSpec((_R, 2), lambda i: (i, 0)),
        out_shape=_f32(N_NODES, 2),
    )(z, agg2, agg2, deg, deg, wp2, bp)


def kernel(node_features, edge_index, edge_src, edge_dst,
           W_self1, W_neigh1, b1, W_self2, W_neigh2, b2, W_pred, b_pred):
    pad = E_PAD - N_EDGES
    pad_src = jnp.arange(pad, dtype=jnp.int32) % N_NODES
    pad_dst = TRASH + jnp.arange(pad, dtype=jnp.int32) % (NPAD - TRASH)
    src = jnp.concatenate([edge_index[0].astype(jnp.int32), pad_src])
    dst = jnp.concatenate([edge_index[1].astype(jnp.int32), pad_dst])
    esrc = edge_src.astype(jnp.int32)
    edst = edge_dst.astype(jnp.int32)

    zf = jnp.zeros((NPAD, F), jnp.float32)
    od = jnp.ones((CHUNK, F), jnp.float32)
    sd = jnp.stack([src.reshape(-1, CHUNK), dst.reshape(-1, CHUNK)], axis=1)
    agg1 = _make_segsum(F, CHUNK)(node_features, sd, zf)
    deg = _make_deg()(dst, zf, od)
    y, z = _layers(node_features, agg1, deg,
                   W_self1, W_neigh1, b1.reshape(1, H),
                   W_self2, W_neigh2, b2.reshape(1, F))
    agg2 = _make_segsum(F, CHUNK)(y, sd, zf)
    st = _scores(z, agg2, deg, W_pred.reshape(2, F), b_pred.reshape(1, 1))
    return _predict(st[:, 0], st[:, 1], esrc, edst)


# speedup vs baseline: 17.9236x; 1.2078x over previous
import dataclasses
import functools

import jax
import jax.numpy as jnp
from jax import lax
from jax.experimental import pallas as pl
from jax.experimental.pallas import tpu as pltpu
from jax.experimental.pallas import tpu_sc as plsc

N_NODES = 10000
N_EDGES = 320000
F = 128
H = 256

NC = 2
NS = 16
NW = NC * NS
L = 16

CHUNK = 128
K_CHUNKS = 80
EDGES_PER_TILE = CHUNK * K_CHUNKS
E_PAD = EDGES_PER_TILE * NW
NPAD = 10112
ROWS_PER_TILE = NPAD // NS
TRASH = N_NODES

@functools.cache
def _vmesh():
    return plsc.VectorSubcoreMesh(core_axis_name="c", subcore_axis_name="s",
                                  num_cores=NC, num_subcores=NS)


def _f32(*shape):
    return jax.ShapeDtypeStruct(shape, jnp.float32)


@functools.cache
def _make_segsum(width, chunk):
    nk = EDGES_PER_TILE // chunk
    scratch = [
        pltpu.VMEM((4, 2, chunk), jnp.int32),
        pltpu.VMEM((2, chunk, width), jnp.float32),
        pltpu.VMEM_SHARED((NPAD, width), jnp.float32),
    ] + [pltpu.SemaphoreType.DMA] * 6

    def body(table_hbm, sd_hbm, zf_hbm, agg_hbm,
             ibuf, rows, acc_sh, g0, g1, i0, i1, i2, i3):
        gsems = (g0, g1)
        isems = (i0, i1, i2, i3)
        c = lax.axis_index("c")
        s = lax.axis_index("s")
        w = c * NS + s
        cg0 = w * nk
        rbase = s * ROWS_PER_TILE

        pltpu.sync_copy(zf_hbm.at[pl.ds(rbase, ROWS_PER_TILE)],
                        acc_sh.at[pl.ds(rbase, ROWS_PER_TILE)])
        plsc.subcore_barrier()

        def start_idx(k, q):
            pltpu.async_copy(sd_hbm.at[cg0 + k], ibuf.at[q], isems[q])

        def wait_idx(q):
            pltpu.make_async_copy(sd_hbm.at[0], ibuf.at[q], isems[q]).wait()

        def start_g(q, b):
            pltpu.async_copy(table_hbm.at[ibuf.at[q, 0]], rows.at[b],
                             gsems[b])

        def drain(q, b):
            pltpu.make_async_copy(table_hbm.at[ibuf.at[q, 0]], rows.at[b],
                                  gsems[b]).wait()
            pltpu.sync_copy(rows.at[b], acc_sh.at[ibuf.at[q, 1]], add=True)

        for q in range(4):
            start_idx(q, q)
        wait_idx(0)
        wait_idx(1)
        start_g(0, 0)
        start_g(1, 1)

        def step(k, j, last):
            drain(j, j % 2)
            if not last or j < 2:
                wait_idx((j + 2) % 4)
                start_g((j + 2) % 4, j % 2)
            if not last:
                start_idx(k + j + 4, j)

        @pl.loop(0, nk - 4, step=4)
        def _(k):
            for j in range(4):
                step(k, j, False)

        for j in range(4):
            step(nk - 4, j, True)
        plsc.subcore_barrier()

        pltpu.sync_copy(acc_sh.at[pl.ds(rbase, ROWS_PER_TILE)],
                        agg_hbm.at[c, pl.ds(rbase, ROWS_PER_TILE)])

    return pl.kernel(body, out_type=_f32(NC, NPAD, width), mesh=_vmesh(),
                     scratch_types=scratch, name=f"segsum{width}")


@functools.cache
def _make_deg():
    def body(dst_hbm, deg_hbm, dbuf, hist, sem):
        c = lax.axis_index("c")
        s = lax.axis_index("s")
        w = c * NS + s
        ebase = w * EDGES_PER_TILE

        pltpu.async_copy(dst_hbm.at[pl.ds(ebase, EDGES_PER_TILE)], dbuf, sem)

        @pl.loop(0, NPAD, step=L)
        def _(i):
            hist[pl.ds(i, L)] = jnp.zeros((L,), jnp.float32)

        pltpu.make_async_copy(dst_hbm.at[pl.ds(0, EDGES_PER_TILE)], dbuf,
                              sem).wait()
        ones = jnp.ones((L,), jnp.float32)

        @pl.loop(0, EDGES_PER_TILE, step=L)
        def _(i):
            plsc.addupdate_scatter(hist, [dbuf[pl.ds(i, L)]], ones)

        pltpu.sync_copy(hist, deg_hbm.at[w])

    cp = pltpu.CompilerParams()
    if "needs_layout_passes" in pltpu.CompilerParams.__dataclass_fields__:
        cp = dataclasses.replace(cp, needs_layout_passes=False)
    return pl.kernel(
        body, out_type=_f32(NW, NPAD), mesh=_vmesh(),
        scratch_types=[
            pltpu.VMEM((EDGES_PER_TILE,), jnp.int32),
            pltpu.VMEM((NPAD,), jnp.float32),
            pltpu.SemaphoreType.DMA,
        ],
        compiler_params=cp,
        name="deg_histogram")


def _predict(sv_hbm_arr, tv_hbm_arr, esrc, edst):
    epw = N_EDGES // NW

    def body(s_hbm, t_hbm, es_hbm, ed_hbm, out_hbm, s_v, t_v, es_v, ed_v,
             out_v):
        c = lax.axis_index("c")
        s = lax.axis_index("s")
        w = c * NS + s
        base = w * epw
        pltpu.sync_copy(s_hbm, s_v)
        pltpu.sync_copy(t_hbm, t_v)
        pltpu.sync_copy(es_hbm.at[pl.ds(base, epw)], es_v)
        pltpu.sync_copy(ed_hbm.at[pl.ds(base, epw)], ed_v)

        @pl.loop(0, epw, step=L)
        def _(i):
            si = es_v[pl.ds(i, L)]
            di = ed_v[pl.ds(i, L)]
            sv = plsc.load_gather(s_v, [si])
            tv = plsc.load_gather(t_v, [di])
            out_v[pl.ds(i, L)] = sv + tv

        pltpu.sync_copy(out_v, out_hbm.at[pl.ds(base, epw)])

    cp = pltpu.CompilerParams()
    if "needs_layout_passes" in pltpu.CompilerParams.__dataclass_fields__:
        cp = dataclasses.replace(cp, needs_layout_passes=False)
    return pl.kernel(
        body, out_type=_f32(N_EDGES), mesh=_vmesh(),
        scratch_types=[
            pltpu.VMEM((N_NODES,), jnp.float32),
            pltpu.VMEM((N_NODES,), jnp.float32),
            pltpu.VMEM((epw,), jnp.int32),
            pltpu.VMEM((epw,), jnp.int32),
            pltpu.VMEM((epw,), jnp.float32),
        ],
        compiler_params=cp,
        name="edge_predict")(sv_hbm_arr, tv_hbm_arr, esrc, edst)


_R = 400
_GRID = N_NODES // _R


def _recip_deg(hist):
    deg = jnp.sum(hist[...], axis=1, keepdims=True)
    return 1.0 / jnp.maximum(deg, 1.0)


def _layer_body(x, a1a, a1b, hist, ws1, wn1, b1, ws2, wn2, b2, y, z):
    mean = (a1a[0] + a1b[0]) * _recip_deg(hist)
    h = jnp.dot(x[...], ws1[...], preferred_element_type=jnp.float32)
    h += jnp.dot(mean, wn1[...], preferred_element_type=jnp.float32)
    h = jnp.maximum(h + b1[0], 0.0)
    y[...] = jnp.dot(h, wn2[...], preferred_element_type=jnp.float32)
    z[...] = (jnp.dot(h, ws2[...], preferred_element_type=jnp.float32)
              + b2[0])


def _layers(x, agg1, hist2, ws1, wn1, b1, ws2, wn2, b2):
    part = lambda core: pl.BlockSpec((1, _R, F), lambda i, c=core: (c, i, 0))
    hpart = pl.BlockSpec((_R, NW), lambda i: (i, 0))
    full = lambda *blk: pl.BlockSpec(blk, lambda i: (0,) * len(blk))
    return pl.pallas_call(
        _layer_body,
        grid=(_GRID,),
        in_specs=[
            pl.BlockSpec((_R, F), lambda i: (i, 0)),
            part(0), part(1), hpart,
            full(F, H), full(F, H), full(1, H),
            full(H, F), full(H, F), full(1, F),
        ],
        out_specs=[pl.BlockSpec((_R, F), lambda i: (i, 0))] * 2,
        out_shape=[_f32(N_NODES, F)] * 2,
    )(x, agg1, agg1, hist2, ws1, wn1, b1, ws2, wn2, b2)


def _score_body(z, a2a, a2b, hist, wp2, bp, st):
    h2 = z[...] + (a2a[0] + a2b[0]) * _recip_deg(hist)
    out = jnp.dot(h2, wp2[...].T, preferred_element_type=jnp.float32)
    is_s = (lax.broadcasted_iota(jnp.int32, (1, 2), 1) == 0)
    st[...] = out + bp[0, 0] * is_s.astype(jnp.float32)


def _scores(z, agg2, hist2, wp2, bp):
    part = lambda core: pl.BlockSpec((1, _R, F), lambda i, c=core: (c, i, 0))
    hpart = pl.BlockSpec((_R, NW), lambda i: (i, 0))
    return pl.pallas_call(
        _score_body,
        grid=(_GRID,),
        in_specs=[
            pl.BlockSpec((_R, F), lambda i: (i, 0)),
            part(0), part(1), hpart,
            pl.BlockSpec((2, F), lambda i: (0, 0)),
            pl.BlockSpec((1, 1), lambda i: (0, 0)),
        ],
        out_specs=pl.BlockSpec((_R, 2), lambda i: (i, 0)),
        out_shape=_f32(N_NODES, 2),
    )(z, agg2, agg2, hist2, wp2, bp)


def kernel(node_features, edge_index, edge_src, edge_dst,
           W_self1, W_neigh1, b1, W_self2, W_neigh2, b2, W_pred, b_pred):
    pad = E_PAD - N_EDGES
    pad_src = jnp.arange(pad, dtype=jnp.int32) % N_NODES
    pad_dst = TRASH + jnp.arange(pad, dtype=jnp.int32) % (NPAD - TRASH)
    src = jnp.concatenate([edge_index[0].astype(jnp.int32), pad_src])
    dst = jnp.concatenate([edge_index[1].astype(jnp.int32), pad_dst])
    esrc = edge_src.astype(jnp.int32)
    edst = edge_dst.astype(jnp.int32)

    zf = jnp.zeros((NPAD, F), jnp.float32)
    sd = jnp.stack([src.reshape(-1, CHUNK), dst.reshape(-1, CHUNK)], axis=1)
    agg1 = _make_segsum(F, CHUNK)(node_features, sd, zf)
    hist2 = _make_deg()(dst).T
    y, z = _layers(node_features, agg1, hist2,
                   W_self1, W_neigh1, b1.reshape(1, H),
                   W_self2, W_neigh2, b2.reshape(1, F))
    agg2 = _make_segsum(F, CHUNK)(y, sd, zf)
    st = _scores(z, agg2, hist2, W_pred.reshape(2, F), b_pred.reshape(1, 1))
    return _predict(st[:, 0], st[:, 1], esrc, edst)
